# baseline (speedup 1.0000x reference)
import jax
import jax.numpy as jnp
from jax.experimental import pallas as pl

N_GRAPHS = 512
BASE_DIM = 128
CHARGE_PAD = 128
NB = 5000


def _tables_kernel(ga_ref, ct_ref, cW1_ref, cb1_ref, cW2_ref, cb2_ref,
                   pW1_ref, C_ref, G_ref):
    f32 = jnp.float32
    pW1_c = pW1_ref[BASE_DIM:BASE_DIM + 16]
    pW1_h = pW1_ref[BASE_DIM + 16:]
    C_ref[...] = jnp.dot(ct_ref[...], pW1_c, preferred_element_type=f32)
    h = jnp.dot(ga_ref[...], cW1_ref[...], preferred_element_type=f32) + cb1_ref[...]
    h = h * jax.nn.sigmoid(h)
    h = jnp.dot(h, cW2_ref[...], preferred_element_type=f32) + cb2_ref[...]
    G_ref[...] = jnp.dot(h, pW1_h, preferred_element_type=f32)


def _fused_kernel(sp_ref, ch_ref, bt_ref, C_ref, G_ref,
                  pW1_ref, pb1_ref, pW2_ref, pb2_ref, out_ref):
    f32 = jnp.float32
    bf16 = jnp.bfloat16
    dn = (((0,), (0,)), ((), ()))
    z = jnp.dot(sp_ref[...], pW1_ref[:BASE_DIM], preferred_element_type=f32)
    oh_c = (ch_ref[0] == jax.lax.broadcasted_iota(jnp.int32, (CHARGE_PAD, NB), 0)).astype(f32)
    z = z + jax.lax.dot_general(oh_c, C_ref[...], dn, preferred_element_type=f32)
    oh_b = (bt_ref[0] == jax.lax.broadcasted_iota(jnp.int32, (N_GRAPHS, NB), 0)).astype(f32)
    z = z + jax.lax.dot_general(oh_b, G_ref[...], dn, preferred_element_type=f32)
    z = z + pb1_ref[...]
    a = z * jax.nn.sigmoid(z)
    out_ref[...] = jnp.dot(a, pW2_ref[...], preferred_element_type=f32) + pb2_ref[...]


def kernel(species_emb, batch, charge, graph_attr, charge_table,
           cW1, cb1, cW2, cb2, pW1, pb1, pW2, pb2):
    n = species_emb.shape[0]
    grid = n // NB
    ch3d = charge.astype(jnp.int32).reshape(grid, 1, NB)
    bt3d = batch.astype(jnp.int32).reshape(grid, 1, NB)
    ct_pad = jnp.zeros((CHARGE_PAD, charge_table.shape[1]), jnp.float32).at[:charge_table.shape[0]].set(charge_table)

    C, G = pl.pallas_call(
        _tables_kernel,
        out_shape=(jax.ShapeDtypeStruct((CHARGE_PAD, BASE_DIM), jnp.float32),
                   jax.ShapeDtypeStruct((N_GRAPHS, BASE_DIM), jnp.float32)),
    )(graph_attr, ct_pad, cW1, cb1.reshape(1, -1), cW2, cb2.reshape(1, -1), pW1)

    full = lambda s: pl.BlockSpec(s, lambda i: (0, 0))
    out = pl.pallas_call(
        _fused_kernel,
        grid=(grid,),
        in_specs=[
            pl.BlockSpec((NB, BASE_DIM), lambda i: (i, 0)),
            pl.BlockSpec((1, 1, NB), lambda i: (i, 0, 0)),
            pl.BlockSpec((1, 1, NB), lambda i: (i, 0, 0)),
            full(C.shape),
            full(G.shape),
            full(pW1.shape),
            full((1, pb1.shape[0])),
            full(pW2.shape),
            full((1, pb2.shape[0])),
        ],
        out_specs=pl.BlockSpec((NB, pW2.shape[1]), lambda i: (i, 0)),
        out_shape=jax.ShapeDtypeStruct((n, pW2.shape[1]), jnp.float32),
    )(species_emb, ch3d, bt3d, C, G,
      pW1, pb1.reshape(1, -1), pW2, pb2.reshape(1, -1))
    return out


# speedup vs baseline: 1.0954x; 1.0954x over previous
import jax
import jax.numpy as jnp
from jax.experimental import pallas as pl

N_GRAPHS = 512
BASE_DIM = 128
CHARGE_PAD = 128
TBL = CHARGE_PAD + N_GRAPHS
NB = 4000


def _tables_kernel(ga_ref, ct_ref, cW1_ref, cb1_ref, cW2_ref, cb2_ref,
                   pW1_ref, pb1_ref, T_ref):
    f32 = jnp.float32
    pW1_c = pW1_ref[BASE_DIM:BASE_DIM + 16]
    pW1_h = pW1_ref[BASE_DIM + 16:]
    T_ref[:CHARGE_PAD] = jnp.dot(ct_ref[...], pW1_c, preferred_element_type=f32) + pb1_ref[...]
    h = jnp.dot(ga_ref[...], cW1_ref[...], preferred_element_type=f32) + cb1_ref[...]
    h = h * jax.nn.sigmoid(h)
    h = jnp.dot(h, cW2_ref[...], preferred_element_type=f32) + cb2_ref[...]
    T_ref[CHARGE_PAD:] = jnp.dot(h, pW1_h, preferred_element_type=f32)


def _fused_kernel(sp_ref, ch_ref, bt_ref, T_ref, pW1_ref, pW2_ref, pb2_ref, out_ref):
    f32 = jnp.float32
    dn = (((0,), (0,)), ((), ()))
    z = jnp.dot(sp_ref[...], pW1_ref[:BASE_DIM], preferred_element_type=f32)
    rows = jax.lax.broadcasted_iota(jnp.int32, (TBL, NB), 0)
    oh = ((rows == ch_ref[0]) | (rows == bt_ref[0])).astype(f32)
    z = z + jax.lax.dot_general(oh, T_ref[...], dn, preferred_element_type=f32)
    a = z * jax.nn.sigmoid(z)
    out_ref[...] = jnp.dot(a, pW2_ref[...], preferred_element_type=f32) + pb2_ref[...]


def kernel(species_emb, batch, charge, graph_attr, charge_table,
           cW1, cb1, cW2, cb2, pW1, pb1, pW2, pb2):
    n = species_emb.shape[0]
    grid = n // NB
    ch3d = charge.astype(jnp.int32).reshape(grid, 1, NB)
    bt3d = (batch.astype(jnp.int32) + CHARGE_PAD).reshape(grid, 1, NB)
    ct_pad = jnp.zeros((CHARGE_PAD, charge_table.shape[1]), jnp.float32).at[:charge_table.shape[0]].set(charge_table)

    T = pl.pallas_call(
        _tables_kernel,
        out_shape=jax.ShapeDtypeStruct((TBL, BASE_DIM), jnp.float32),
    )(graph_attr, ct_pad, cW1, cb1.reshape(1, -1), cW2, cb2.reshape(1, -1),
      pW1, pb1.reshape(1, -1))

    full = lambda s: pl.BlockSpec(s, lambda i: (0, 0))
    out = pl.pallas_call(
        _fused_kernel,
        grid=(grid,),
        in_specs=[
            pl.BlockSpec((NB, BASE_DIM), lambda i: (i, 0)),
            pl.BlockSpec((1, 1, NB), lambda i: (i, 0, 0)),
            pl.BlockSpec((1, 1, NB), lambda i: (i, 0, 0)),
            full(T.shape),
            full(pW1.shape),
            full(pW2.shape),
            full((1, pb2.shape[0])),
        ],
        out_specs=pl.BlockSpec((NB, pW2.shape[1]), lambda i: (i, 0)),
        out_shape=jax.ShapeDtypeStruct((n, pW2.shape[1]), jnp.float32),
    )(species_emb, ch3d, bt3d, T, pW1, pW2, pb2.reshape(1, -1))
    return out


# speedup vs baseline: 1.1789x; 1.0763x over previous
import jax
import jax.numpy as jnp
from jax.experimental import pallas as pl

N_GRAPHS = 512
BASE_DIM = 128
CHARGE_PAD = 128
TBL = CHARGE_PAD + N_GRAPHS
NB = 4000


def _tables_kernel(ga_ref, ct_ref, cW1_ref, cb1_ref, cW2_ref, cb2_ref,
                   pW1_ref, pb1_ref, T_ref):
    f32 = jnp.float32
    pW1_c = pW1_ref[BASE_DIM:BASE_DIM + 16]
    pW1_h = pW1_ref[BASE_DIM + 16:]
    T_ref[:CHARGE_PAD] = jnp.dot(ct_ref[...], pW1_c, preferred_element_type=f32) + pb1_ref[...]
    h = jnp.dot(ga_ref[...], cW1_ref[...], preferred_element_type=f32) + cb1_ref[...]
    h = h * jax.nn.sigmoid(h)
    h = jnp.dot(h, cW2_ref[...], preferred_element_type=f32) + cb2_ref[...]
    T_ref[CHARGE_PAD:] = jnp.dot(h, pW1_h, preferred_element_type=f32)


def _fused_kernel(sp_ref, ch_ref, bt_ref, T_ref, pW1_ref, pW2_ref, pb2_ref, out_ref):
    f32 = jnp.float32
    dn = (((0,), (0,)), ((), ()))
    z = jnp.dot(sp_ref[...], pW1_ref[:BASE_DIM], preferred_element_type=f32)
    oh_c = (ch_ref[0] == jax.lax.broadcasted_iota(jnp.int32, (CHARGE_PAD, NB), 0)).astype(f32)
    z = z + jax.lax.dot_general(oh_c, T_ref[:CHARGE_PAD], dn, preferred_element_type=f32)
    oh_b = (bt_ref[0] == jax.lax.broadcasted_iota(jnp.int32, (N_GRAPHS, NB), 0)).astype(f32)
    z = z + jax.lax.dot_general(oh_b, T_ref[CHARGE_PAD:], dn, preferred_element_type=f32)
    a = z * jax.nn.sigmoid(z)
    out_ref[...] = jnp.dot(a, pW2_ref[...], preferred_element_type=f32) + pb2_ref[...]


def kernel(species_emb, batch, charge, graph_attr, charge_table,
           cW1, cb1, cW2, cb2, pW1, pb1, pW2, pb2):
    n = species_emb.shape[0]
    grid = n // NB
    ch3d = charge.astype(jnp.int32).reshape(grid, 1, NB)
    bt3d = batch.astype(jnp.int32).reshape(grid, 1, NB)
    ct_pad = jnp.zeros((CHARGE_PAD, charge_table.shape[1]), jnp.float32).at[:charge_table.shape[0]].set(charge_table)

    T = pl.pallas_call(
        _tables_kernel,
        out_shape=jax.ShapeDtypeStruct((TBL, BASE_DIM), jnp.float32),
    )(graph_attr, ct_pad, cW1, cb1.reshape(1, -1), cW2, cb2.reshape(1, -1),
      pW1, pb1.reshape(1, -1))

    full = lambda s: pl.BlockSpec(s, lambda i: (0, 0))
    out = pl.pallas_call(
        _fused_kernel,
        grid=(grid,),
        in_specs=[
            pl.BlockSpec((NB, BASE_DIM), lambda i: (i, 0)),
            pl.BlockSpec((1, 1, NB), lambda i: (i, 0, 0)),
            pl.BlockSpec((1, 1, NB), lambda i: (i, 0, 0)),
            full(T.shape),
            full(pW1.shape),
            full(pW2.shape),
            full((1, pb2.shape[0])),
        ],
        out_specs=pl.BlockSpec((NB, pW2.shape[1]), lambda i: (i, 0)),
        out_shape=jax.ShapeDtypeStruct((n, pW2.shape[1]), jnp.float32),
    )(species_emb, ch3d, bt3d, T, pW1, pW2, pb2.reshape(1, -1))
    return out
